# baseline (speedup 1.0000x reference)
import functools

import jax
import jax.numpy as jnp
from jax import lax
from jax.experimental import pallas as pl
from jax.experimental.pallas import tpu as pltpu
from jax.experimental.pallas import tpu_sc as plsc

_LANES = 16


@functools.lru_cache(maxsize=None)
def _make_sc_call(NW, NCHUNK, C, S, E, V):
    mesh = plsc.VectorSubcoreMesh(core_axis_name="c", subcore_axis_name="s")
    NC = 2

    @functools.partial(
        pl.kernel,
        mesh=mesh,
        compiler_params=pltpu.CompilerParams(use_tc_tiling_on_sc=False),
        out_type=jax.ShapeDtypeStruct((NW * NCHUNK * C, E), jnp.float32),
        scratch_types=[
            pltpu.VMEM((C,), jnp.int32),
            pltpu.VMEM((C,), jnp.int32),
            pltpu.VMEM((S, E), jnp.float32),
            pltpu.VMEM((C, E), jnp.float32),
            pltpu.VMEM((C, E), jnp.float32),
            pltpu.SemaphoreType.DMA,
            pltpu.SemaphoreType.DMA,
            pltpu.SemaphoreType.DMA,
            pltpu.SemaphoreType.DMA,
            pltpu.SemaphoreType.DMA,
            pltpu.SemaphoreType.DMA,
        ],
    )
    def sc_call(x_hbm, tok_hbm, pos_hbm, out_hbm,
                idx0, idx1, pos_v, rows0, rows1,
                ig0, ig1, g0, g1, s0, s1):
        wid = lax.axis_index("s") * NC + lax.axis_index("c")
        base = wid * (NCHUNK * C)

        pltpu.sync_copy(pos_hbm, pos_v)

        idxs = (idx0, idx1)
        bufs = (rows0, rows1)
        igs = (ig0, ig1)
        gs = (g0, g1)
        ss = (s0, s1)

        def add_pos(buf):
            for q in range(C // S):
                def body(j, _):
                    r = q * S + j
                    for k in range(E // _LANES):
                        sl = pl.ds(k * _LANES, _LANES)
                        buf[r, sl] = buf[r, sl] + pos_v[j, sl]
                    return 0
                lax.fori_loop(0, S, body, 0, unroll=2)

        def idx_copy(c):
            return pltpu.async_copy(x_hbm.at[wid, c], idxs[c % 2], igs[c % 2])

        icopies = [None] * NCHUNK
        gathers = [None] * NCHUNK
        scatters = [None] * NCHUNK

        icopies[0] = idx_copy(0)
        if NCHUNK > 1:
            icopies[1] = idx_copy(1)
        icopies[0].wait()
        gathers[0] = pltpu.async_copy(tok_hbm.at[idx0], bufs[0], gs[0])

        for c in range(NCHUNK):
            b = c % 2
            nb = (c + 1) % 2
            gathers[c].wait()
            if c + 2 < NCHUNK:
                icopies[c + 2] = idx_copy(c + 2)
            if c + 1 < NCHUNK:
                icopies[c + 1].wait()
                if c >= 1:
                    scatters[c - 1].wait()
                gathers[c + 1] = pltpu.async_copy(
                    tok_hbm.at[idxs[nb]], bufs[nb], gs[nb])
            add_pos(bufs[b])
            scatters[c] = pltpu.async_copy(
                bufs[b], out_hbm.at[pl.ds(base + c * C, C)], ss[b])

        if NCHUNK > 1:
            scatters[NCHUNK - 2].wait()
        scatters[NCHUNK - 1].wait()

    return sc_call


def kernel(x, token_table, pos_table):
    B, S = x.shape
    V, E = token_table.shape
    N = B * S

    NW = 32
    C = 2 * S
    NCHUNK = N // (NW * C)
    assert NW * NCHUNK * C == N and C % S == 0 and E % _LANES == 0

    xf = x.reshape(N).astype(jnp.int32).reshape(NW, NCHUNK, C)
    pos = pos_table[:S]
    call = _make_sc_call(NW, NCHUNK, C, S, E, V)
    out = call(xf, token_table, pos)
    return out.reshape(B, S, E)
